# baseline (speedup 1.0000x reference)
import functools

import jax
import jax.numpy as jnp
import numpy as np
from jax import lax
from jax.experimental import pallas as pl
from jax.experimental.pallas import tpu as pltpu
from jax.experimental.pallas import tpu_sc as plsc

B = 16384
L = 50
DIM = 32
VOCAB = 81616

NC = 2
NS = 16
NW = NC * NS

GPC = 4
CLEN = GPC * L
NCHUNK = B // GPC
CPW = NCHUNK // NW
RPW = B // NW
NBUF = 4
G1 = 128
G2 = CLEN - G1

TBLK = 128
NFULL = VOCAB // TBLK
TAIL = VOCAB - NFULL * TBLK
TAIL_SID = NFULL % NS
BPT = NFULL // NS + 1


def _tree_sum(vs):
    while len(vs) > 1:
        nxt = [a + b for a, b in zip(vs[0::2], vs[1::2])]
        if len(vs) % 2:
            nxt.append(vs[-1])
        vs = nxt
    return vs[0]


def _pack_block(slab, tbuf, width):
    for i in range(width):
        ga = slab[i, pl.ds(0, 16)]
        gb = slab[i, pl.ds(16, 16)]
        tbuf[i] = plsc.pack(ga, gb, format=plsc.PackFormat.INTERLEAVED)


def _sc_body(w_hbm, idx_hbm, out_hbm, table_sh, acc_v, slab_v, tbuf_v,
             tail_slab, tail_tbuf, *bufs):
    idx_bufs = bufs[0:NBUF]
    row_bufs = bufs[NBUF:2 * NBUF]
    isems, rsems = bufs[2 * NBUF], bufs[2 * NBUF + 1]
    ssems, osems = bufs[2 * NBUF + 2], bufs[2 * NBUF + 3]

    cid = lax.axis_index("c")
    sid = lax.axis_index("s")
    wid = sid * NC + cid
    chunk0 = wid * CPW

    def bid_of(i):
        return i * NS + sid

    def start_slab(i, u):
        @pl.when(bid_of(i) < NFULL)
        def _():
            pltpu.async_copy(
                w_hbm.at[pl.ds(bid_of(i) * TBLK, TBLK)], slab_v.at[u],
                ssems.at[u])

    def wait_slab(u):
        pltpu.make_async_copy(
            w_hbm.at[pl.ds(0, TBLK)], slab_v.at[u], ssems.at[u]).wait()

    start_slab(0, 0)
    start_slab(1, 1)

    def stage_body(t, carry):
        for u in range(2):
            i = 2 * t + u

            @pl.when(bid_of(i) < NFULL)
            def _():
                wait_slab(u)

                @pl.when(t > 0)
                def _():
                    pltpu.make_async_copy(
                        tbuf_v.at[u], table_sh.at[pl.ds(0, TBLK)],
                        osems.at[u]).wait()

                _pack_block(slab_v.at[u], tbuf_v.at[u], TBLK)
                pltpu.async_copy(
                    tbuf_v.at[u], table_sh.at[pl.ds(bid_of(i) * TBLK, TBLK)],
                    osems.at[u])

            start_slab(i + 2, u)
        return carry

    lax.fori_loop(0, BPT // 2, stage_body, 0)

    for u in range(2):
        @pl.when(bid_of(BPT - 2 + u) < NFULL)
        def _():
            pltpu.make_async_copy(
                tbuf_v.at[u], table_sh.at[pl.ds(0, TBLK)], osems.at[u]).wait()

    @pl.when(sid == TAIL_SID)
    def _():
        pltpu.sync_copy(w_hbm.at[pl.ds(NFULL * TBLK, TAIL)], tail_slab)
        _pack_block(tail_slab, tail_tbuf, TAIL)
        pltpu.sync_copy(tail_tbuf, table_sh.at[pl.ds(NFULL * TBLK, TAIL)])

    plsc.subcore_barrier()

    def start_idx(k, b):
        pltpu.async_copy(
            idx_hbm.at[pl.ds((chunk0 + k) * CLEN, CLEN)], idx_bufs[b], isems.at[b])

    def wait_idx(b):
        pltpu.make_async_copy(
            idx_hbm.at[pl.ds(0, CLEN)], idx_bufs[b], isems.at[b]).wait()

    def start_gather(b):
        pltpu.async_copy(
            table_sh.at[idx_bufs[b].at[pl.ds(0, G1)]],
            row_bufs[b].at[pl.ds(0, G1)], rsems.at[b])
        pltpu.async_copy(
            table_sh.at[idx_bufs[b].at[pl.ds(G1, G2)]],
            row_bufs[b].at[pl.ds(G1, G2)], rsems.at[b])

    def wait_rows(b):
        pltpu.make_async_copy(
            table_sh.at[idx_bufs[0]], row_bufs[b], rsems.at[b]).wait()

    for b in range(NBUF):
        start_idx(b, b)
    for b in range(NBUF):
        wait_idx(b)
        start_gather(b)

    def accumulate(k, b):
        for g in range(GPC):
            s = _tree_sum([row_bufs[b][g * L + r] for r in range(L)])
            v0, v1 = plsc.unpack(
                s, format=plsc.PackFormat.INTERLEAVED,
                preferred_element_type=jnp.float32)
            acc_v[pl.ds((k * GPC + g) * DIM, 16)] = v0
            acc_v[pl.ds((k * GPC + g) * DIM + 16, 16)] = v1

    def ring_body(j, carry):
        for b in range(NBUF):
            k = NBUF * j + b
            wait_rows(b)

            @pl.when(k + NBUF < CPW)
            def _():
                start_idx(k + NBUF, b)

            accumulate(k, b)

            @pl.when(k + NBUF < CPW)
            def _():
                wait_idx(b)
                start_gather(b)

        return carry

    lax.fori_loop(0, CPW // NBUF, ring_body, 0)

    pltpu.sync_copy(acc_v, out_hbm.at[pl.ds(wid * RPW * DIM, RPW * DIM)])


@jax.jit
def _sc_embed_sum(w_rows, idx_flat):
    mesh = plsc.VectorSubcoreMesh(core_axis_name="c", subcore_axis_name="s")
    scratch = [
        pltpu.VMEM_SHARED((VOCAB, DIM), jnp.bfloat16),
        pltpu.VMEM((RPW * DIM,), jnp.float32),
        pltpu.VMEM((2, TBLK, DIM), jnp.float32),
        pltpu.VMEM((2, TBLK, DIM), jnp.bfloat16),
        pltpu.VMEM((TAIL, DIM), jnp.float32),
        pltpu.VMEM((TAIL, DIM), jnp.bfloat16),
    ]
    scratch += [pltpu.VMEM((CLEN,), jnp.int32) for _ in range(NBUF)]
    scratch += [pltpu.VMEM((CLEN, DIM), jnp.bfloat16) for _ in range(NBUF)]
    scratch += [pltpu.SemaphoreType.DMA((NBUF,)), pltpu.SemaphoreType.DMA((NBUF,)),
                pltpu.SemaphoreType.DMA((2,)), pltpu.SemaphoreType.DMA((2,))]
    return pl.kernel(
        _sc_body,
        out_type=jax.ShapeDtypeStruct((B * DIM,), jnp.float32),
        mesh=mesh,
        scratch_types=scratch,
        compiler_params=pltpu.CompilerParams(
            use_tc_tiling_on_sc=False, needs_layout_passes=False),
    )(w_rows, idx_flat)


def kernel(inputs, w):
    idx_flat = inputs.astype(jnp.int32).reshape(B * L)
    out_flat = _sc_embed_sum(w, idx_flat)
    return out_flat.reshape(B, DIM)


# speedup vs baseline: 1.0617x; 1.0617x over previous
import functools

import jax
import jax.numpy as jnp
import numpy as np
from jax import lax
from jax.experimental import pallas as pl
from jax.experimental.pallas import tpu as pltpu
from jax.experimental.pallas import tpu_sc as plsc

B = 16384
L = 50
DIM = 32
VOCAB = 81616

NC = 2
NS = 16
NW = NC * NS

GPC = 4
CLEN = GPC * L
NCHUNK = B // GPC
CPW = NCHUNK // NW
RPW = B // NW
NBUF = 4
G1 = 128
G2 = CLEN - G1

TBLK = 128
NFULL = VOCAB // TBLK
TAIL = VOCAB - NFULL * TBLK
TAIL_SID = NFULL % NS
BPT = NFULL // NS + 1


def _tree_sum(vs):
    while len(vs) > 1:
        nxt = [a + b for a, b in zip(vs[0::2], vs[1::2])]
        if len(vs) % 2:
            nxt.append(vs[-1])
        vs = nxt
    return vs[0]


def _pack_block(slab, tbuf, width):
    for i in range(width):
        ga = slab[i, pl.ds(0, 16)]
        gb = slab[i, pl.ds(16, 16)]
        tbuf[i] = plsc.pack(ga, gb, format=plsc.PackFormat.INTERLEAVED)


def _sc_body(w_hbm, idx_hbm, out_hbm, table_sh, acc_v, slab_v, tbuf_v,
             tail_slab, tail_tbuf, *bufs):
    idx_bufs = bufs[0:NBUF]
    row_bufs = bufs[NBUF:2 * NBUF]
    isems, rsems = bufs[2 * NBUF], bufs[2 * NBUF + 1]
    ssems, osems = bufs[2 * NBUF + 2], bufs[2 * NBUF + 3]

    cid = lax.axis_index("c")
    sid = lax.axis_index("s")
    wid = sid * NC + cid
    chunk0 = wid * CPW

    def bid_of(i):
        return i * NS + sid

    def start_slab(i, u):
        @pl.when(bid_of(i) < NFULL)
        def _():
            pltpu.async_copy(
                w_hbm.at[pl.ds(bid_of(i) * TBLK, TBLK)], slab_v.at[u],
                ssems.at[u])

    def wait_slab(u):
        pltpu.make_async_copy(
            w_hbm.at[pl.ds(0, TBLK)], slab_v.at[u], ssems.at[u]).wait()

    start_slab(0, 0)
    start_slab(1, 1)

    def stage_body(t, carry):
        for u in range(2):
            i = 2 * t + u

            @pl.when(bid_of(i) < NFULL)
            def _():
                wait_slab(u)

                @pl.when(t > 0)
                def _():
                    pltpu.make_async_copy(
                        tbuf_v.at[u], table_sh.at[pl.ds(0, TBLK)],
                        osems.at[u]).wait()

                _pack_block(slab_v.at[u], tbuf_v.at[u], TBLK)
                pltpu.async_copy(
                    tbuf_v.at[u], table_sh.at[pl.ds(bid_of(i) * TBLK, TBLK)],
                    osems.at[u])

            start_slab(i + 2, u)
        return carry

    lax.fori_loop(0, BPT // 2, stage_body, 0)

    for u in range(2):
        @pl.when(bid_of(BPT - 2 + u) < NFULL)
        def _():
            pltpu.make_async_copy(
                tbuf_v.at[u], table_sh.at[pl.ds(0, TBLK)], osems.at[u]).wait()

    @pl.when(sid == TAIL_SID)
    def _():
        pltpu.sync_copy(w_hbm.at[pl.ds(NFULL * TBLK, TAIL)], tail_slab)
        _pack_block(tail_slab, tail_tbuf, TAIL)
        pltpu.sync_copy(tail_tbuf, table_sh.at[pl.ds(NFULL * TBLK, TAIL)])

    plsc.subcore_barrier()

    def start_idx(k, b):
        pltpu.async_copy(
            idx_hbm.at[pl.ds((chunk0 + k) * CLEN, CLEN)], idx_bufs[b], isems.at[b])

    def wait_idx(b):
        pltpu.make_async_copy(
            idx_hbm.at[pl.ds(0, CLEN)], idx_bufs[b], isems.at[b]).wait()

    def start_gather(b):
        pltpu.async_copy(
            table_sh.at[idx_bufs[b].at[pl.ds(0, G1)]],
            row_bufs[b].at[pl.ds(0, G1)], rsems.at[b])
        pltpu.async_copy(
            table_sh.at[idx_bufs[b].at[pl.ds(G1, G2)]],
            row_bufs[b].at[pl.ds(G1, G2)], rsems.at[b])

    def wait_rows(b):
        pltpu.make_async_copy(
            table_sh.at[idx_bufs[0]], row_bufs[b], rsems.at[b]).wait()

    for b in range(NBUF):
        start_idx(b, b)
    for b in range(NBUF):
        wait_idx(b)
        start_gather(b)

    lanes = lax.iota(jnp.int32, 16)

    def accumulate(k, b):
        for g in range(GPC):
            s = _tree_sum([row_bufs[b][g * L + r] for r in range(L)])
            v0, v1 = plsc.unpack(
                s, format=plsc.PackFormat.INTERLEAVED,
                preferred_element_type=jnp.float32)
            col = jnp.full((16,), k * GPC + g, jnp.int32)
            plsc.store_scatter(acc_v, [lanes, col], v0)
            plsc.store_scatter(acc_v, [lanes + 16, col], v1)

    def ring_body(j, carry):
        for b in range(NBUF):
            k = NBUF * j + b
            wait_rows(b)

            @pl.when(k + NBUF < CPW)
            def _():
                start_idx(k + NBUF, b)

            accumulate(k, b)

            @pl.when(k + NBUF < CPW)
            def _():
                wait_idx(b)
                start_gather(b)

        return carry

    lax.fori_loop(0, CPW // NBUF, ring_body, 0)

    pltpu.sync_copy(acc_v, out_hbm.at[:, pl.ds(wid * RPW, RPW)])


@jax.jit
def _sc_embed_sum(w_rows, idx_flat):
    mesh = plsc.VectorSubcoreMesh(core_axis_name="c", subcore_axis_name="s")
    scratch = [
        pltpu.VMEM_SHARED((VOCAB, DIM), jnp.bfloat16),
        pltpu.VMEM((DIM, RPW), jnp.float32),
        pltpu.VMEM((2, TBLK, DIM), jnp.float32),
        pltpu.VMEM((2, TBLK, DIM), jnp.bfloat16),
        pltpu.VMEM((TAIL, DIM), jnp.float32),
        pltpu.VMEM((TAIL, DIM), jnp.bfloat16),
    ]
    scratch += [pltpu.VMEM((CLEN,), jnp.int32) for _ in range(NBUF)]
    scratch += [pltpu.VMEM((CLEN, DIM), jnp.bfloat16) for _ in range(NBUF)]
    scratch += [pltpu.SemaphoreType.DMA((NBUF,)), pltpu.SemaphoreType.DMA((NBUF,)),
                pltpu.SemaphoreType.DMA((2,)), pltpu.SemaphoreType.DMA((2,))]
    return pl.kernel(
        _sc_body,
        out_type=jax.ShapeDtypeStruct((DIM, B), jnp.float32),
        mesh=mesh,
        scratch_types=scratch,
        compiler_params=pltpu.CompilerParams(
            use_tc_tiling_on_sc=False, needs_layout_passes=False),
    )(w_rows, idx_flat)


def kernel(inputs, w):
    idx_flat = inputs.astype(jnp.int32).reshape(B * L)
    out_t = _sc_embed_sum(w, idx_flat)
    return out_t.T


# speedup vs baseline: 1.2782x; 1.2038x over previous
import functools

import jax
import jax.numpy as jnp
import numpy as np
from jax import lax
from jax.experimental import pallas as pl
from jax.experimental.pallas import tpu as pltpu
from jax.experimental.pallas import tpu_sc as plsc

B = 16384
L = 50
DIM = 32
VOCAB = 81616

NC = 2
NS = 16
NW = NC * NS

GPC = 8
CLEN = GPC * L
NCHUNK = B // GPC
CPW = NCHUNK // NW
RPW = B // NW
NBUF = 2
GMAX = 128
GSPLITS = [(o, min(GMAX, CLEN - o)) for o in range(0, CLEN, GMAX)]

TBLK = 128
NFULL = VOCAB // TBLK
TAIL = VOCAB - NFULL * TBLK
TAIL_SID = NFULL % NS
BPT = NFULL // NS + 1


def _tree_sum(vs):
    while len(vs) > 1:
        nxt = [a + b for a, b in zip(vs[0::2], vs[1::2])]
        if len(vs) % 2:
            nxt.append(vs[-1])
        vs = nxt
    return vs[0]


def _pack_block(slab, tbuf, width):
    for i in range(width):
        ga = slab[i, pl.ds(0, 16)]
        gb = slab[i, pl.ds(16, 16)]
        tbuf[i] = plsc.pack(ga, gb, format=plsc.PackFormat.INTERLEAVED)


def _sc_body(w_hbm, idx_hbm, out_hbm, table_sh, acc_v, slab_v, tbuf_v,
             tail_slab, tail_tbuf, *bufs):
    idx_bufs = bufs[0:NBUF]
    row_bufs = bufs[NBUF:2 * NBUF]
    isems, rsems = bufs[2 * NBUF], bufs[2 * NBUF + 1]
    ssems, osems = bufs[2 * NBUF + 2], bufs[2 * NBUF + 3]

    cid = lax.axis_index("c")
    sid = lax.axis_index("s")
    wid = sid * NC + cid
    chunk0 = wid * CPW

    def bid_of(i):
        return i * NS + sid

    def start_slab(i, u):
        @pl.when(bid_of(i) < NFULL)
        def _():
            pltpu.async_copy(
                w_hbm.at[pl.ds(bid_of(i) * TBLK, TBLK)], slab_v.at[u],
                ssems.at[u])

    def wait_slab(u):
        pltpu.make_async_copy(
            w_hbm.at[pl.ds(0, TBLK)], slab_v.at[u], ssems.at[u]).wait()

    start_slab(0, 0)
    start_slab(1, 1)

    def stage_body(t, carry):
        for u in range(2):
            i = 2 * t + u

            @pl.when(bid_of(i) < NFULL)
            def _():
                wait_slab(u)

                @pl.when(t > 0)
                def _():
                    pltpu.make_async_copy(
                        tbuf_v.at[u], table_sh.at[pl.ds(0, TBLK)],
                        osems.at[u]).wait()

                _pack_block(slab_v.at[u], tbuf_v.at[u], TBLK)
                pltpu.async_copy(
                    tbuf_v.at[u], table_sh.at[pl.ds(bid_of(i) * TBLK, TBLK)],
                    osems.at[u])

            start_slab(i + 2, u)
        return carry

    lax.fori_loop(0, BPT // 2, stage_body, 0)

    for u in range(2):
        @pl.when(bid_of(BPT - 2 + u) < NFULL)
        def _():
            pltpu.make_async_copy(
                tbuf_v.at[u], table_sh.at[pl.ds(0, TBLK)], osems.at[u]).wait()

    @pl.when(sid == TAIL_SID)
    def _():
        pltpu.sync_copy(w_hbm.at[pl.ds(NFULL * TBLK, TAIL)], tail_slab)
        _pack_block(tail_slab, tail_tbuf, TAIL)
        pltpu.sync_copy(tail_tbuf, table_sh.at[pl.ds(NFULL * TBLK, TAIL)])

    plsc.subcore_barrier()

    def start_idx(k, b):
        pltpu.async_copy(
            idx_hbm.at[pl.ds((chunk0 + k) * CLEN, CLEN)], idx_bufs[b], isems.at[b])

    def wait_idx(b):
        pltpu.make_async_copy(
            idx_hbm.at[pl.ds(0, CLEN)], idx_bufs[b], isems.at[b]).wait()

    def start_gather(b):
        for o, n in GSPLITS:
            pltpu.async_copy(
                table_sh.at[idx_bufs[b].at[pl.ds(o, n)]],
                row_bufs[b].at[pl.ds(o, n)], rsems.at[b])

    def wait_rows(b):
        pltpu.make_async_copy(
            table_sh.at[idx_bufs[0]], row_bufs[b], rsems.at[b]).wait()

    for b in range(NBUF):
        start_idx(b, b)
    for b in range(NBUF):
        wait_idx(b)
        start_gather(b)

    lanes = lax.iota(jnp.int32, 16)

    def accumulate(k, b):
        for g in range(GPC):
            s = _tree_sum([row_bufs[b][g * L + r] for r in range(L)])
            v0, v1 = plsc.unpack(
                s, format=plsc.PackFormat.INTERLEAVED,
                preferred_element_type=jnp.float32)
            col = jnp.full((16,), k * GPC + g, jnp.int32)
            plsc.store_scatter(acc_v, [lanes, col], v0)
            plsc.store_scatter(acc_v, [lanes + 16, col], v1)

    def ring_body(j, carry):
        for b in range(NBUF):
            k = NBUF * j + b
            wait_rows(b)

            @pl.when(k + NBUF < CPW)
            def _():
                start_idx(k + NBUF, b)

            accumulate(k, b)

            @pl.when(k + NBUF < CPW)
            def _():
                wait_idx(b)
                start_gather(b)

        return carry

    lax.fori_loop(0, CPW // NBUF, ring_body, 0)

    pltpu.sync_copy(acc_v, out_hbm.at[:, pl.ds(wid * RPW, RPW)])


@jax.jit
def _sc_embed_sum(w_rows, idx_flat):
    mesh = plsc.VectorSubcoreMesh(core_axis_name="c", subcore_axis_name="s")
    scratch = [
        pltpu.VMEM_SHARED((VOCAB, DIM), jnp.bfloat16),
        pltpu.VMEM((DIM, RPW), jnp.float32),
        pltpu.VMEM((2, TBLK, DIM), jnp.float32),
        pltpu.VMEM((2, TBLK, DIM), jnp.bfloat16),
        pltpu.VMEM((TAIL, DIM), jnp.float32),
        pltpu.VMEM((TAIL, DIM), jnp.bfloat16),
    ]
    scratch += [pltpu.VMEM((CLEN,), jnp.int32) for _ in range(NBUF)]
    scratch += [pltpu.VMEM((CLEN, DIM), jnp.bfloat16) for _ in range(NBUF)]
    scratch += [pltpu.SemaphoreType.DMA((NBUF,)), pltpu.SemaphoreType.DMA((NBUF,)),
                pltpu.SemaphoreType.DMA((2,)), pltpu.SemaphoreType.DMA((2,))]
    return pl.kernel(
        _sc_body,
        out_type=jax.ShapeDtypeStruct((DIM, B), jnp.float32),
        mesh=mesh,
        scratch_types=scratch,
        compiler_params=pltpu.CompilerParams(
            use_tc_tiling_on_sc=False, needs_layout_passes=False),
    )(w_rows, idx_flat)


def kernel(inputs, w):
    idx_flat = inputs.astype(jnp.int32).reshape(B * L)
    out_t = _sc_embed_sum(w, idx_flat)
    return out_t.T
